# Initial kernel scaffold; baseline (speedup 1.0000x reference)
import jax
import jax.numpy as jnp
from jax.experimental import pallas as pl


def kernel(verts, edges, faces, faces_to_edges):
    raise NotImplementedError("write your pallas kernel here")


# speedup vs baseline: 3.7859x; 3.7859x over previous
import functools

import jax
import jax.numpy as jnp
from jax import lax
from jax.experimental import pallas as pl
from jax.experimental.pallas import tpu as pltpu
from jax.experimental.pallas import tpu_sc as plsc

N_CORES = 2
N_SUBCORES = 16
N_TILES = N_CORES * N_SUBCORES
LANES = 16

C_EDGE = 1200
C_FACE = 1000
C_FACE_PAD = 1008
E_OFF3 = 3 * C_FACE_PAD


def _make_kernel(N, V, E, F):
    echunks_per_batch = E // C_EDGE
    n_fchunks = F // C_FACE

    @functools.partial(
        pl.kernel,
        mesh=plsc.VectorSubcoreMesh(core_axis_name="c", subcore_axis_name="s"),
        out_type=(
            jax.ShapeDtypeStruct((N * E, 3), jnp.float32),
            jax.ShapeDtypeStruct((N * 4 * F * 3,), jnp.int32),
        ),
        compiler_params=pltpu.CompilerParams(needs_layout_passes=False,
                                             use_tc_tiling_on_sc=False),
        scratch_types=[
            pltpu.VMEM((2 * C_EDGE,), jnp.int32),
            pltpu.VMEM((2 * C_EDGE,), jnp.int32),
            pltpu.VMEM((2 * C_EDGE, 8), jnp.float32),
            pltpu.VMEM((C_EDGE, 3), jnp.float32),
            pltpu.VMEM((2 * E_OFF3,), jnp.int32),
            pltpu.VMEM((3 * C_FACE_PAD,), jnp.int32),
        ],
    )
    def body(verts4, edges_f, faces_f, f2e_f, mids, nf, ebuf, idxb,
             rall, mbuf, cb, obuf):
        cid = lax.axis_index("c")
        sid = lax.axis_index("s")
        wid = sid * N_CORES + cid

        iota = lax.iota(jnp.int32, LANES)

        for n in range(N):
            nbase = n * V

            def echunk(t, _, nbase=nbase, n=n):
                q = wid + N_TILES * t

                @pl.when(q < echunks_per_batch)
                def _():
                    e0 = q * C_EDGE
                    pltpu.sync_copy(edges_f.at[pl.ds(2 * e0, 2 * C_EDGE)],
                                    ebuf)

                    def build_idx(t2, _):
                        g = plsc.load_gather(ebuf, [iota + LANES * t2])
                        idxb[pl.ds(LANES * t2, LANES)] = g + nbase
                        return 0

                    lax.fori_loop(0, (2 * C_EDGE) // LANES, build_idx, 0,
                                  unroll=False)

                    pltpu.sync_copy(verts4.at[idxb], rall)

                    def combine(t3, _):
                        p = iota + LANES * t3
                        e = p // 3
                        c = p - 3 * e
                        a = plsc.load_gather(rall, [2 * e, c])
                        b = plsc.load_gather(rall, [2 * e + 1, c])
                        plsc.store_scatter(mbuf, [e, c], (a + b) * 0.5)
                        return 0

                    lax.fori_loop(0, (3 * C_EDGE) // LANES, combine, 0,
                                  unroll=False)
                    pltpu.sync_copy(mbuf, mids.at[pl.ds(n * E + e0, C_EDGE)])

                return 0

            lax.fori_loop(0, -(-echunks_per_batch // N_TILES), echunk, 0,
                          unroll=False)

        def fchunk(t, _):
            fc = wid + N_TILES * t

            @pl.when(fc < n_fchunks)
            def _():
                f0 = fc * C_FACE
                pltpu.sync_copy(faces_f.at[pl.ds(3 * f0, 3 * C_FACE)],
                                cb.at[pl.ds(0, 3 * C_FACE)])
                pltpu.sync_copy(f2e_f.at[pl.ds(3 * f0, 3 * C_FACE)],
                                cb.at[pl.ds(E_OFF3, 3 * C_FACE)])

                nblk = -(-(3 * C_FACE) // LANES)
                for s in range(4):
                    def emit(t3, _, s=s):
                        p = iota + LANES * t3
                        e3 = p - (p % 3)
                        c = p - e3
                        if s < 3:
                            is_f = c == 0
                            col = jnp.where(is_f, s,
                                            jnp.where(c == 1, (s + 2) % 3,
                                                      (s + 1) % 3))
                            src = e3 + col + jnp.where(is_f, 0, E_OFF3)
                            addv = jnp.where(is_f, 0, V)
                        else:
                            src = e3 + c + E_OFF3
                            addv = V
                        val = plsc.load_gather(cb, [src]) + addv
                        obuf[pl.ds(LANES * t3, LANES)] = val
                        return 0

                    lax.fori_loop(0, nblk, emit, 0, unroll=False)
                    for n in range(N):
                        pltpu.sync_copy(
                            obuf.at[pl.ds(0, 3 * C_FACE)],
                            nf.at[pl.ds(3 * (n * 4 * F + s * F + f0),
                                        3 * C_FACE)])

            return 0

        lax.fori_loop(0, -(-n_fchunks // N_TILES), fchunk, 0, unroll=False)

    return body


def kernel(verts, edges, faces, faces_to_edges):
    N, V, _ = verts.shape
    E = edges.shape[0]
    F = faces.shape[0]
    k = _make_kernel(N, V, E, F)
    verts4 = jnp.pad(verts.reshape(N * V, 3), ((0, 0), (0, 5)))
    mids, nf = k(verts4, edges.reshape(-1), faces.reshape(-1),
                 faces_to_edges.reshape(-1))
    new_verts = jnp.concatenate([verts, mids.reshape(N, E, 3)], axis=1)
    return new_verts, nf.reshape(N, 4 * F, 3)


# speedup vs baseline: 3.7897x; 1.0010x over previous
import functools

import jax
import jax.numpy as jnp
from jax import lax
from jax.experimental import pallas as pl
from jax.experimental.pallas import tpu as pltpu
from jax.experimental.pallas import tpu_sc as plsc

N_CORES = 2
N_SUBCORES = 16
N_TILES = N_CORES * N_SUBCORES
LANES = 16

C_EDGE = 1200
C_FACE = 1000
C_FACE_PAD = 1008
E_OFF3 = 3 * C_FACE_PAD


def _make_kernel(N, V, E, F):
    echunks_per_batch = E // C_EDGE
    n_fchunks = F // C_FACE

    @functools.partial(
        pl.kernel,
        mesh=plsc.VectorSubcoreMesh(core_axis_name="c", subcore_axis_name="s"),
        out_type=(
            jax.ShapeDtypeStruct((N * E, 3), jnp.float32),
            jax.ShapeDtypeStruct((N * 4 * F * 3,), jnp.int32),
        ),
        compiler_params=pltpu.CompilerParams(needs_layout_passes=False,
                                             use_tc_tiling_on_sc=False),
        scratch_types=[
            pltpu.VMEM((2 * C_EDGE,), jnp.int32),
            pltpu.VMEM((2 * C_EDGE,), jnp.int32),
            pltpu.VMEM((2 * C_EDGE, 8), jnp.float32),
            pltpu.VMEM((C_EDGE, 3), jnp.float32),
            pltpu.VMEM((2 * E_OFF3,), jnp.int32),
            pltpu.VMEM((3 * C_FACE_PAD,), jnp.int32),
        ],
    )
    def body(verts4, edges_f, faces_f, f2e_f, mids, nf, ebuf, idxb,
             rall, mbuf, cb, obuf):
        cid = lax.axis_index("c")
        sid = lax.axis_index("s")
        wid = sid * N_CORES + cid

        iota = lax.iota(jnp.int32, LANES)

        for n in range(N):
            nbase = n * V

            def echunk(t, _, nbase=nbase, n=n):
                q = wid + N_TILES * t

                @pl.when(q < echunks_per_batch)
                def _():
                    e0 = q * C_EDGE
                    pltpu.sync_copy(edges_f.at[pl.ds(2 * e0, 2 * C_EDGE)],
                                    ebuf)

                    def build_idx(t2, _):
                        g = plsc.load_gather(ebuf, [iota + LANES * t2])
                        idxb[pl.ds(LANES * t2, LANES)] = g + nbase
                        return 0

                    lax.fori_loop(0, (2 * C_EDGE) // LANES, build_idx, 0,
                                  unroll=8)

                    pltpu.sync_copy(verts4.at[idxb], rall)

                    def combine(t3, _):
                        p = iota + LANES * t3
                        e = p // 3
                        c = p - 3 * e
                        a = plsc.load_gather(rall, [2 * e, c])
                        b = plsc.load_gather(rall, [2 * e + 1, c])
                        plsc.store_scatter(mbuf, [e, c], (a + b) * 0.5)
                        return 0

                    lax.fori_loop(0, (3 * C_EDGE) // LANES, combine, 0,
                                  unroll=8)
                    pltpu.sync_copy(mbuf, mids.at[pl.ds(n * E + e0, C_EDGE)])

                return 0

            lax.fori_loop(0, -(-echunks_per_batch // N_TILES), echunk, 0,
                          unroll=False)

        def fchunk(t, _):
            fc = wid + N_TILES * t

            @pl.when(fc < n_fchunks)
            def _():
                f0 = fc * C_FACE
                pltpu.sync_copy(faces_f.at[pl.ds(3 * f0, 3 * C_FACE)],
                                cb.at[pl.ds(0, 3 * C_FACE)])
                pltpu.sync_copy(f2e_f.at[pl.ds(3 * f0, 3 * C_FACE)],
                                cb.at[pl.ds(E_OFF3, 3 * C_FACE)])

                nblk = -(-(3 * C_FACE) // LANES)
                for s in range(4):
                    def emit(t3, _, s=s):
                        p = iota + LANES * t3
                        e3 = p - (p % 3)
                        c = p - e3
                        if s < 3:
                            is_f = c == 0
                            col = jnp.where(is_f, s,
                                            jnp.where(c == 1, (s + 2) % 3,
                                                      (s + 1) % 3))
                            src = e3 + col + jnp.where(is_f, 0, E_OFF3)
                            addv = jnp.where(is_f, 0, V)
                        else:
                            src = e3 + c + E_OFF3
                            addv = V
                        val = plsc.load_gather(cb, [src]) + addv
                        obuf[pl.ds(LANES * t3, LANES)] = val
                        return 0

                    lax.fori_loop(0, nblk, emit, 0, unroll=4)
                    for n in range(N):
                        pltpu.sync_copy(
                            obuf.at[pl.ds(0, 3 * C_FACE)],
                            nf.at[pl.ds(3 * (n * 4 * F + s * F + f0),
                                        3 * C_FACE)])

            return 0

        lax.fori_loop(0, -(-n_fchunks // N_TILES), fchunk, 0, unroll=False)

    return body


def kernel(verts, edges, faces, faces_to_edges):
    N, V, _ = verts.shape
    E = edges.shape[0]
    F = faces.shape[0]
    k = _make_kernel(N, V, E, F)
    verts4 = jnp.pad(verts.reshape(N * V, 3), ((0, 0), (0, 5)))
    mids, nf = k(verts4, edges.reshape(-1), faces.reshape(-1),
                 faces_to_edges.reshape(-1))
    new_verts = jnp.concatenate([verts, mids.reshape(N, E, 3)], axis=1)
    return new_verts, nf.reshape(N, 4 * F, 3)
